# baseline (speedup 1.0000x reference)
import functools
import math

import jax
import jax.numpy as jnp
from jax import lax
from jax.experimental import pallas as pl
from jax.experimental.pallas import tpu as pltpu
from jax.experimental.pallas import tpu_sc as plsc

N = 10000
E = 160000
D_IN = 256
D_H = 128
NL = 3
NT = 2

NSC = 2
NSUB = 16
HALF = N // NSC
HALF_PAD = 5120
CPR = HALF_PAD // NSUB
ROWS_PAD = NT * HALF_PAD
ZROWS = ROWS_PAD // NSUB
EC = E // NSUB
CH = 128
NBUF = 2
SIB = 4
LIB = 2
NCHUNK = 80
NCHP = NCHUNK + SIB
ECP = NCHUNK * CH

BLK = 1000
NXP = 11000
NBP = NXP // BLK
ZR = N
BN_SCALE = 1.0 / math.sqrt(1.0 + 1e-5)

_mesh = plsc.VectorSubcoreMesh(core_axis_name="c", subcore_axis_name="s")


@functools.partial(
    pl.kernel,
    mesh=_mesh,
    out_type=jax.ShapeDtypeStruct((NSC, NT, HALF_PAD, D_H), jnp.float32),
    scratch_types=[
        pltpu.VMEM((SIB, 1, CH), jnp.int32),
        pltpu.VMEM((LIB, 1, CH), jnp.int32),
        pltpu.VMEM((NBUF, CH, D_H), jnp.float32),
        pltpu.VMEM_SHARED((ROWS_PAD, D_H), jnp.float32),
        pltpu.SemaphoreType.DMA((SIB,)),
        pltpu.SemaphoreType.DMA((LIB,)),
        pltpu.SemaphoreType.DMA((NBUF,)),
    ],
)
def _sc_aggregate(h_hbm, src_hbm, lidx_hbm, zeros_hbm, out_hbm,
                  siv, liv, rows_v, ash, sem_si, sem_li, sem_g):
    c = lax.axis_index("c")
    s = lax.axis_index("s")
    pltpu.sync_copy(zeros_hbm, ash.at[pl.ds(s * ZROWS, ZROWS)])
    plsc.subcore_barrier()

    def si_start(j):
        b = lax.rem(j, SIB)
        pltpu.async_copy(src_hbm.at[c, s, j], siv.at[b], sem_si.at[b])

    def si_wait(j):
        b = lax.rem(j, SIB)
        pltpu.make_async_copy(src_hbm.at[c, s, j], siv.at[b],
                              sem_si.at[b]).wait()

    def li_start(j):
        b = lax.rem(j, LIB)
        pltpu.async_copy(lidx_hbm.at[c, s, j], liv.at[b], sem_li.at[b])

    def li_wait(j):
        b = lax.rem(j, LIB)
        pltpu.make_async_copy(lidx_hbm.at[c, s, j], liv.at[b],
                              sem_li.at[b]).wait()

    def g_start(j):
        b = lax.rem(j, NBUF)
        pltpu.async_copy(h_hbm.at[siv.at[lax.rem(j, SIB), 0]], rows_v.at[b],
                         sem_g.at[b])

    def g_wait(j):
        b = lax.rem(j, NBUF)
        pltpu.make_async_copy(h_hbm.at[siv.at[lax.rem(j, SIB), 0]],
                              rows_v.at[b], sem_g.at[b]).wait()

    lax.fori_loop(0, SIB, lambda j, cr: (si_start(j), cr)[1], 0)
    lax.fori_loop(0, LIB, lambda j, cr: (li_start(j), cr)[1], 0)
    lax.fori_loop(0, NBUF,
                  lambda j, cr: (si_wait(j), g_start(j), cr)[-1], 0)

    def body(j, carry):
        li_wait(j)
        g_wait(j)
        b = lax.rem(j, NBUF)
        pltpu.sync_copy(rows_v.at[b], ash.at[liv.at[lax.rem(j, LIB), 0]],
                        add=True)
        li_start(j + LIB)
        si_wait(j + NBUF)
        g_start(j + NBUF)
        si_start(j + SIB)
        return carry

    lax.fori_loop(0, NCHUNK, body, 0)
    lax.fori_loop(NCHUNK, NCHUNK + NBUF, lambda j, cr: (g_wait(j), cr)[1], 0)
    lax.fori_loop(NCHUNK + NBUF, NCHUNK + SIB,
                  lambda j, cr: (si_wait(j), cr)[1], 0)
    lax.fori_loop(NCHUNK, NCHUNK + LIB, lambda j, cr: (li_wait(j), cr)[1], 0)
    plsc.subcore_barrier()
    for t in range(NT):
        pltpu.sync_copy(ash.at[pl.ds(t * HALF_PAD + s * CPR, CPR)],
                        out_hbm.at[c, t, pl.ds(s * CPR, CPR)])


def _dot(a, b):
    return lax.dot_general(a, b, (((1,), (1,)), ((), ())),
                           preferred_element_type=jnp.float32)


def _tc_proj_body(x_ref, wp_ref, bp_ref, out_ref):
    i = pl.program_id(0)

    @pl.when(i < NBP - 1)
    def _():
        out_ref[...] = _dot(x_ref[...], wp_ref[...]) + bp_ref[...]

    @pl.when(i == NBP - 1)
    def _():
        out_ref[...] = jnp.zeros((BLK, D_H), jnp.float32)


def _tc_proj(x, wp, bp):
    return pl.pallas_call(
        _tc_proj_body,
        grid=(NBP,),
        in_specs=[
            pl.BlockSpec((BLK, D_IN), lambda i: (i, 0)),
            pl.BlockSpec((D_H, D_IN), lambda i: (0, 0)),
            pl.BlockSpec((1, D_H), lambda i: (0, 0)),
        ],
        out_specs=pl.BlockSpec((BLK, D_H), lambda i: (i, 0)),
        out_shape=jax.ShapeDtypeStruct((NXP, D_H), jnp.float32),
    )(x, wp, bp)


def _tc_layer_body(h_ref, a0_ref, a1_ref, c0_ref, c1_ref,
                   we0_ref, we1_ref, ws_ref,
                   be0_ref, be1_ref, bs_ref, gm_ref, bt_ref,
                   out_ref, *, bn):
    i = pl.program_id(0)

    @pl.when(i < NBP - 1)
    def _():
        acc = (_dot(a0_ref[0, 0], we0_ref[...]) +
               _dot(a1_ref[0, 0], we1_ref[...]) +
               _dot(h_ref[...], ws_ref[...]))
        c0 = c0_ref[0, 0][:, 0:1]
        c1 = c1_ref[0, 0][:, 0:1]
        acc = acc + c0 * be0_ref[...] + c1 * be1_ref[...] + bs_ref[...]
        deg = c0 + c1
        deg = jnp.where(deg == 0.0, 1.0, deg)
        h = jnp.maximum(acc / deg, 0.0)
        if bn:
            h = h * (gm_ref[...] * BN_SCALE) + bt_ref[...]
            h = jnp.maximum(h, 0.0)
        out_ref[...] = h

    @pl.when(i == NBP - 1)
    def _():
        out_ref[...] = jnp.zeros((BLK, D_H), jnp.float32)


_NBH = HALF // BLK


def _tc_layer(h, agg, cnt, lp, bnp):
    bn = bnp is not None
    gm = bnp['gamma'] if bn else lp['bs']
    bt = bnp['beta'] if bn else lp['bs']
    row = lambda v: v.reshape(1, D_H)
    full = lambda: pl.BlockSpec((D_H, D_H), lambda i: (0, 0))
    vec = lambda: pl.BlockSpec((1, D_H), lambda i: (0, 0))
    piece = lambda t: pl.BlockSpec(
        (1, 1, BLK, D_H),
        lambda i, t=t: (jnp.minimum(i // _NBH, NSC - 1), t,
                        jnp.minimum(i % _NBH, _NBH - 1), 0))
    return pl.pallas_call(
        functools.partial(_tc_layer_body, bn=bn),
        grid=(NBP,),
        in_specs=[pl.BlockSpec((BLK, D_H), lambda i: (i, 0)),
                  piece(0), piece(1), piece(0), piece(1),
                  full(), full(), full(),
                  vec(), vec(), vec(), vec(), vec()],
        out_specs=pl.BlockSpec((BLK, D_H), lambda i: (i, 0)),
        out_shape=jax.ShapeDtypeStruct((NXP, D_H), jnp.float32),
    )(h, agg, agg, cnt, cnt,
      lp['We'][0], lp['We'][1], lp['Ws'],
      row(lp['be'][0]), row(lp['be'][1]), row(lp['bs']), row(gm), row(bt))


def _tc_head_body(h_ref, wr1_ref, br1_ref, wr2_ref, br2_ref, out_ref, acc_ref):
    i = pl.program_id(0)

    @pl.when(i == 0)
    def _():
        acc_ref[...] = jnp.zeros((8, D_H), jnp.float32)

    acc_ref[...] = acc_ref[...] + jnp.sum(h_ref[...], axis=0, keepdims=True)

    @pl.when(i == NBP - 1)
    def _():
        g = acc_ref[0:1, :] * (1.0 / N)
        z = jnp.maximum(_dot(g, wr1_ref[...]) + br1_ref[...], 0.0)
        out_ref[...] = _dot(z, wr2_ref[...]) + br2_ref[...]


def _tc_head(h, wr1, br1, wr2, br2):
    return pl.pallas_call(
        _tc_head_body,
        grid=(NBP,),
        in_specs=[
            pl.BlockSpec((BLK, D_H), lambda i: (i, 0)),
            pl.BlockSpec((D_H, D_H), lambda i: (0, 0)),
            pl.BlockSpec((1, D_H), lambda i: (0, 0)),
            pl.BlockSpec((D_H, D_H), lambda i: (0, 0)),
            pl.BlockSpec((1, D_H), lambda i: (0, 0)),
        ],
        out_specs=pl.BlockSpec((1, D_H), lambda i: (0, 0)),
        out_shape=jax.ShapeDtypeStruct((1, D_H), jnp.float32),
        scratch_shapes=[pltpu.VMEM((8, D_H), jnp.float32)],
    )(h, wr1, br1.reshape(1, D_H), wr2, br2.reshape(1, D_H))


def kernel(x, edge_index, edge_types, params):
    src = edge_index[0].astype(jnp.int32)
    dst = edge_index[1].astype(jnp.int32)
    et = edge_types.astype(jnp.int32)

    owner = dst // HALF
    lrow = et * HALF_PAD + (dst % HALF)
    srcs = jnp.stack([jnp.where(owner == c, src, ZR) for c in range(NSC)])
    lidx = jnp.stack([jnp.where(owner == c, lrow, 0) for c in range(NSC)])

    srcw = jnp.pad(srcs.reshape(NSC, NSUB, EC),
                   ((0, 0), (0, 0), (0, NCHP * CH - EC)),
                   constant_values=ZR
                   ).reshape(NSC, NSUB, NCHP, 1, CH)
    lidxw = jnp.pad(lidx.reshape(NSC, NSUB, EC),
                    ((0, 0), (0, 0), (0, NCHP * CH - EC))
                    ).reshape(NSC, NSUB, NCHP, 1, CH)

    zeros_big = jnp.zeros((ZROWS, D_H), jnp.float32)
    h_ones = jnp.concatenate([jnp.ones((N, D_H), jnp.float32),
                              jnp.zeros((NXP - N, D_H), jnp.float32)])

    cnt = _sc_aggregate(h_ones, srcw, lidxw, zeros_big)

    p = params
    xp = jnp.pad(x, ((0, NXP - N), (0, 0)))
    h = _tc_proj(xp, p['Wp'], p['bp'].reshape(1, D_H))
    for i in range(NL):
        agg = _sc_aggregate(h, srcw, lidxw, zeros_big)
        bnp = p['bn'][i] if i < NL - 1 else None
        h = _tc_layer(h, agg, cnt, p['layers'][i], bnp)
    return _tc_head(h, p['Wr1'], p['br1'], p['Wr2'], p['br2'])


# speedup vs baseline: 1.0501x; 1.0501x over previous
import functools
import math

import jax
import jax.numpy as jnp
from jax import lax
from jax.experimental import pallas as pl
from jax.experimental.pallas import tpu as pltpu
from jax.experimental.pallas import tpu_sc as plsc

N = 10000
E = 160000
D_IN = 256
D_H = 128
NL = 3
NT = 2

NSC = 2
NSUB = 16
HALF = N // NSC
HALF_PAD = 5120
CPR = HALF_PAD // NSUB
ROWS_PAD = NT * HALF_PAD
ZROWS = ROWS_PAD // NSUB
EC = E // NSUB
CH = 128
NBUF = 2
SIB = 4
LIB = 2
NCHUNK = 80
NCHP = NCHUNK + SIB
ECP = NCHUNK * CH

BLK = 1000
NXP = 11000
NBP = NXP // BLK
ZR = N
BN_SCALE = 1.0 / math.sqrt(1.0 + 1e-5)

_mesh = plsc.VectorSubcoreMesh(core_axis_name="c", subcore_axis_name="s")


@functools.partial(
    pl.kernel,
    mesh=_mesh,
    out_type=jax.ShapeDtypeStruct((NSC, NT, HALF_PAD, D_H), jnp.float32),
    scratch_types=[
        pltpu.VMEM((NCHP, 1, CH), jnp.int32),
        pltpu.VMEM((NCHP, 1, CH), jnp.int32),
        pltpu.VMEM((CH, D_H), jnp.float32),
        pltpu.VMEM_SHARED((ROWS_PAD, D_H), jnp.float32),
        pltpu.SemaphoreType.DMA,
    ],
)
def _sc_aggregate(h_hbm, src_hbm, lidx_hbm, zeros_hbm, out_hbm,
                  src_v, lidx_v, rows_v, ash, sem):
    c = lax.axis_index("c")
    s = lax.axis_index("s")
    pltpu.sync_copy(zeros_hbm, ash.at[pl.ds(s * ZROWS, ZROWS)])
    pltpu.sync_copy(src_hbm.at[c, s], src_v)
    pltpu.sync_copy(lidx_hbm.at[c, s], lidx_v)
    plsc.subcore_barrier()

    def body(j, carry):
        pltpu.async_copy(h_hbm.at[src_v.at[j, 0]], rows_v, sem).wait()
        pltpu.sync_copy(rows_v, ash.at[lidx_v.at[j, 0]], add=True)
        return carry

    lax.fori_loop(0, NCHUNK, body, 0)
    plsc.subcore_barrier()
    for t in range(NT):
        pltpu.sync_copy(ash.at[pl.ds(t * HALF_PAD + s * CPR, CPR)],
                        out_hbm.at[c, t, pl.ds(s * CPR, CPR)])


def _dot(a, b):
    return lax.dot_general(a, b, (((1,), (1,)), ((), ())),
                           preferred_element_type=jnp.float32)


def _tc_proj_body(x_ref, wp_ref, bp_ref, out_ref):
    i = pl.program_id(0)

    @pl.when(i < NBP - 1)
    def _():
        out_ref[...] = _dot(x_ref[...], wp_ref[...]) + bp_ref[...]

    @pl.when(i == NBP - 1)
    def _():
        out_ref[...] = jnp.zeros((BLK, D_H), jnp.float32)


def _tc_proj(x, wp, bp):
    return pl.pallas_call(
        _tc_proj_body,
        grid=(NBP,),
        in_specs=[
            pl.BlockSpec((BLK, D_IN), lambda i: (i, 0)),
            pl.BlockSpec((D_H, D_IN), lambda i: (0, 0)),
            pl.BlockSpec((1, D_H), lambda i: (0, 0)),
        ],
        out_specs=pl.BlockSpec((BLK, D_H), lambda i: (i, 0)),
        out_shape=jax.ShapeDtypeStruct((NXP, D_H), jnp.float32),
    )(x, wp, bp)


def _tc_layer_body(h_ref, a0_ref, a1_ref, c0_ref, c1_ref,
                   we0_ref, we1_ref, ws_ref,
                   be0_ref, be1_ref, bs_ref, gm_ref, bt_ref,
                   out_ref, *, bn):
    i = pl.program_id(0)

    @pl.when(i < NBP - 1)
    def _():
        acc = (_dot(a0_ref[0, 0], we0_ref[...]) +
               _dot(a1_ref[0, 0], we1_ref[...]) +
               _dot(h_ref[...], ws_ref[...]))
        c0 = c0_ref[0, 0][:, 0:1]
        c1 = c1_ref[0, 0][:, 0:1]
        acc = acc + c0 * be0_ref[...] + c1 * be1_ref[...] + bs_ref[...]
        deg = c0 + c1
        deg = jnp.where(deg == 0.0, 1.0, deg)
        h = jnp.maximum(acc / deg, 0.0)
        if bn:
            h = h * (gm_ref[...] * BN_SCALE) + bt_ref[...]
            h = jnp.maximum(h, 0.0)
        out_ref[...] = h

    @pl.when(i == NBP - 1)
    def _():
        out_ref[...] = jnp.zeros((BLK, D_H), jnp.float32)


_NBH = HALF // BLK


def _tc_layer(h, agg, cnt, lp, bnp):
    bn = bnp is not None
    gm = bnp['gamma'] if bn else lp['bs']
    bt = bnp['beta'] if bn else lp['bs']
    row = lambda v: v.reshape(1, D_H)
    full = lambda: pl.BlockSpec((D_H, D_H), lambda i: (0, 0))
    vec = lambda: pl.BlockSpec((1, D_H), lambda i: (0, 0))
    piece = lambda t: pl.BlockSpec(
        (1, 1, BLK, D_H),
        lambda i, t=t: (jnp.minimum(i // _NBH, NSC - 1), t,
                        jnp.minimum(i % _NBH, _NBH - 1), 0))
    return pl.pallas_call(
        functools.partial(_tc_layer_body, bn=bn),
        grid=(NBP,),
        in_specs=[pl.BlockSpec((BLK, D_H), lambda i: (i, 0)),
                  piece(0), piece(1), piece(0), piece(1),
                  full(), full(), full(),
                  vec(), vec(), vec(), vec(), vec()],
        out_specs=pl.BlockSpec((BLK, D_H), lambda i: (i, 0)),
        out_shape=jax.ShapeDtypeStruct((NXP, D_H), jnp.float32),
    )(h, agg, agg, cnt, cnt,
      lp['We'][0], lp['We'][1], lp['Ws'],
      row(lp['be'][0]), row(lp['be'][1]), row(lp['bs']), row(gm), row(bt))


def _tc_head_body(h_ref, wr1_ref, br1_ref, wr2_ref, br2_ref, out_ref, acc_ref):
    i = pl.program_id(0)

    @pl.when(i == 0)
    def _():
        acc_ref[...] = jnp.zeros((8, D_H), jnp.float32)

    acc_ref[...] = acc_ref[...] + jnp.sum(h_ref[...], axis=0, keepdims=True)

    @pl.when(i == NBP - 1)
    def _():
        g = acc_ref[0:1, :] * (1.0 / N)
        z = jnp.maximum(_dot(g, wr1_ref[...]) + br1_ref[...], 0.0)
        out_ref[...] = _dot(z, wr2_ref[...]) + br2_ref[...]


def _tc_head(h, wr1, br1, wr2, br2):
    return pl.pallas_call(
        _tc_head_body,
        grid=(NBP,),
        in_specs=[
            pl.BlockSpec((BLK, D_H), lambda i: (i, 0)),
            pl.BlockSpec((D_H, D_H), lambda i: (0, 0)),
            pl.BlockSpec((1, D_H), lambda i: (0, 0)),
            pl.BlockSpec((D_H, D_H), lambda i: (0, 0)),
            pl.BlockSpec((1, D_H), lambda i: (0, 0)),
        ],
        out_specs=pl.BlockSpec((1, D_H), lambda i: (0, 0)),
        out_shape=jax.ShapeDtypeStruct((1, D_H), jnp.float32),
        scratch_shapes=[pltpu.VMEM((8, D_H), jnp.float32)],
    )(h, wr1, br1.reshape(1, D_H), wr2, br2.reshape(1, D_H))


def kernel(x, edge_index, edge_types, params):
    src = edge_index[0].astype(jnp.int32)
    dst = edge_index[1].astype(jnp.int32)
    et = edge_types.astype(jnp.int32)

    owner = dst // HALF
    lrow = et * HALF_PAD + (dst % HALF)
    srcs = jnp.stack([jnp.where(owner == c, src, ZR) for c in range(NSC)])
    lidx = jnp.stack([jnp.where(owner == c, lrow, 0) for c in range(NSC)])
    ow = jnp.stack([(owner == c).astype(jnp.int32) for c in range(NSC)])

    def chunked(a, pad_value):
        return jnp.pad(a.reshape(NSC, NSUB, EC),
                       ((0, 0), (0, 0), (0, NCHP * CH - EC)),
                       constant_values=pad_value
                       ).reshape(NSC, NSUB, NCHP, 1, CH)

    srcw = chunked(srcs, ZR)
    lidxw = chunked(lidx, 0)
    oww = chunked(ow, 0)

    zeros_big = jnp.zeros((ZROWS, D_H), jnp.float32)
    ones2 = jnp.concatenate([jnp.zeros((1, D_H), jnp.float32),
                             jnp.ones((1, D_H), jnp.float32),
                             jnp.zeros((6, D_H), jnp.float32)])

    cnt = _sc_aggregate(ones2, oww, lidxw, zeros_big)

    p = params
    xp = jnp.pad(x, ((0, NXP - N), (0, 0)))
    h = _tc_proj(xp, p['Wp'], p['bp'].reshape(1, D_H))
    for i in range(NL):
        agg = _sc_aggregate(h, srcw, lidxw, zeros_big)
        bnp = p['bn'][i] if i < NL - 1 else None
        h = _tc_layer(h, agg, cnt, p['layers'][i], bnp)
    return _tc_head(h, p['Wr1'], p['br1'], p['Wr2'], p['br2'])


# speedup vs baseline: 1.0502x; 1.0001x over previous
import functools
import math

import jax
import jax.numpy as jnp
from jax import lax
from jax.experimental import pallas as pl
from jax.experimental.pallas import tpu as pltpu
from jax.experimental.pallas import tpu_sc as plsc

N = 10000
E = 160000
D_IN = 256
D_H = 128
NL = 3
NT = 2

NSC = 2
NSUB = 16
HALF = N // NSC
HALF_PAD = 5120
CPR = HALF_PAD // NSUB
ROWS_PAD = NT * HALF_PAD
ZROWS = ROWS_PAD // NSUB
EC = E // NSUB
CH = 128
NBUF = 2
SIB = 4
LIB = 2
NCHUNK = 80
NCHP = NCHUNK + SIB
ECP = NCHUNK * CH

BLK = 1000
NXP = 11000
NBP = NXP // BLK
ZR = N
BN_SCALE = 1.0 / math.sqrt(1.0 + 1e-5)

_mesh = plsc.VectorSubcoreMesh(core_axis_name="c", subcore_axis_name="s")


@functools.partial(
    pl.kernel,
    mesh=_mesh,
    out_type=jax.ShapeDtypeStruct((NSC, NT, HALF_PAD, D_H), jnp.float32),
    scratch_types=[
        pltpu.VMEM((NCHP, CH), jnp.int32),
        pltpu.VMEM((NCHP, CH), jnp.int32),
        pltpu.VMEM((CH, D_H), jnp.float32),
        pltpu.VMEM_SHARED((ROWS_PAD, D_H), jnp.float32),
        pltpu.SemaphoreType.DMA,
    ],
)
def _sc_aggregate(h_hbm, src_hbm, lidx_hbm, zeros_hbm, out_hbm,
                  src_v, lidx_v, rows_v, ash, sem):
    c = lax.axis_index("c")
    s = lax.axis_index("s")
    pltpu.sync_copy(zeros_hbm, ash.at[pl.ds(s * ZROWS, ZROWS)])
    pltpu.sync_copy(src_hbm.at[c, s], src_v)
    pltpu.sync_copy(lidx_hbm.at[c, s], lidx_v)
    plsc.subcore_barrier()

    def body(j, carry):
        pltpu.async_copy(h_hbm.at[src_v.at[j]], rows_v, sem).wait()
        pltpu.sync_copy(rows_v, ash.at[lidx_v.at[j]], add=True)
        return carry

    lax.fori_loop(0, NCHUNK, body, 0)
    plsc.subcore_barrier()
    for t in range(NT):
        pltpu.sync_copy(ash.at[pl.ds(t * HALF_PAD + s * CPR, CPR)],
                        out_hbm.at[c, t, pl.ds(s * CPR, CPR)])


def _dot(a, b):
    return lax.dot_general(a, b, (((1,), (1,)), ((), ())),
                           preferred_element_type=jnp.float32)


def _tc_proj_body(x_ref, wp_ref, bp_ref, out_ref):
    i = pl.program_id(0)

    @pl.when(i < NBP - 1)
    def _():
        out_ref[...] = _dot(x_ref[...], wp_ref[...]) + bp_ref[...]

    @pl.when(i == NBP - 1)
    def _():
        out_ref[...] = jnp.zeros((BLK, D_H), jnp.float32)


def _tc_proj(x, wp, bp):
    return pl.pallas_call(
        _tc_proj_body,
        grid=(NBP,),
        in_specs=[
            pl.BlockSpec((BLK, D_IN), lambda i: (i, 0)),
            pl.BlockSpec((D_H, D_IN), lambda i: (0, 0)),
            pl.BlockSpec((1, D_H), lambda i: (0, 0)),
        ],
        out_specs=pl.BlockSpec((BLK, D_H), lambda i: (i, 0)),
        out_shape=jax.ShapeDtypeStruct((NXP, D_H), jnp.float32),
    )(x, wp, bp)


def _tc_layer_body(h_ref, a0_ref, a1_ref, c0_ref, c1_ref,
                   we0_ref, we1_ref, ws_ref,
                   be0_ref, be1_ref, bs_ref, gm_ref, bt_ref,
                   out_ref, *, bn):
    i = pl.program_id(0)

    @pl.when(i < NBP - 1)
    def _():
        acc = (_dot(a0_ref[0, 0], we0_ref[...]) +
               _dot(a1_ref[0, 0], we1_ref[...]) +
               _dot(h_ref[...], ws_ref[...]))
        c0 = c0_ref[0, 0][:, 0:1]
        c1 = c1_ref[0, 0][:, 0:1]
        acc = acc + c0 * be0_ref[...] + c1 * be1_ref[...] + bs_ref[...]
        deg = c0 + c1
        deg = jnp.where(deg == 0.0, 1.0, deg)
        h = jnp.maximum(acc / deg, 0.0)
        if bn:
            h = h * (gm_ref[...] * BN_SCALE) + bt_ref[...]
            h = jnp.maximum(h, 0.0)
        out_ref[...] = h

    @pl.when(i == NBP - 1)
    def _():
        out_ref[...] = jnp.zeros((BLK, D_H), jnp.float32)


_NBH = HALF // BLK


def _tc_layer(h, agg, cnt, lp, bnp):
    bn = bnp is not None
    gm = bnp['gamma'] if bn else lp['bs']
    bt = bnp['beta'] if bn else lp['bs']
    row = lambda v: v.reshape(1, D_H)
    full = lambda: pl.BlockSpec((D_H, D_H), lambda i: (0, 0))
    vec = lambda: pl.BlockSpec((1, D_H), lambda i: (0, 0))
    piece = lambda t: pl.BlockSpec(
        (1, 1, BLK, D_H),
        lambda i, t=t: (jnp.minimum(i // _NBH, NSC - 1), t,
                        jnp.minimum(i % _NBH, _NBH - 1), 0))
    return pl.pallas_call(
        functools.partial(_tc_layer_body, bn=bn),
        grid=(NBP,),
        in_specs=[pl.BlockSpec((BLK, D_H), lambda i: (i, 0)),
                  piece(0), piece(1), piece(0), piece(1),
                  full(), full(), full(),
                  vec(), vec(), vec(), vec(), vec()],
        out_specs=pl.BlockSpec((BLK, D_H), lambda i: (i, 0)),
        out_shape=jax.ShapeDtypeStruct((NXP, D_H), jnp.float32),
    )(h, agg, agg, cnt, cnt,
      lp['We'][0], lp['We'][1], lp['Ws'],
      row(lp['be'][0]), row(lp['be'][1]), row(lp['bs']), row(gm), row(bt))


def _tc_head_body(h_ref, wr1_ref, br1_ref, wr2_ref, br2_ref, out_ref, acc_ref):
    i = pl.program_id(0)

    @pl.when(i == 0)
    def _():
        acc_ref[...] = jnp.zeros((8, D_H), jnp.float32)

    acc_ref[...] = acc_ref[...] + jnp.sum(h_ref[...], axis=0, keepdims=True)

    @pl.when(i == NBP - 1)
    def _():
        g = acc_ref[0:1, :] * (1.0 / N)
        z = jnp.maximum(_dot(g, wr1_ref[...]) + br1_ref[...], 0.0)
        out_ref[...] = _dot(z, wr2_ref[...]) + br2_ref[...]


def _tc_head(h, wr1, br1, wr2, br2):
    return pl.pallas_call(
        _tc_head_body,
        grid=(NBP,),
        in_specs=[
            pl.BlockSpec((BLK, D_H), lambda i: (i, 0)),
            pl.BlockSpec((D_H, D_H), lambda i: (0, 0)),
            pl.BlockSpec((1, D_H), lambda i: (0, 0)),
            pl.BlockSpec((D_H, D_H), lambda i: (0, 0)),
            pl.BlockSpec((1, D_H), lambda i: (0, 0)),
        ],
        out_specs=pl.BlockSpec((1, D_H), lambda i: (0, 0)),
        out_shape=jax.ShapeDtypeStruct((1, D_H), jnp.float32),
        scratch_shapes=[pltpu.VMEM((8, D_H), jnp.float32)],
    )(h, wr1, br1.reshape(1, D_H), wr2, br2.reshape(1, D_H))


def kernel(x, edge_index, edge_types, params):
    src = edge_index[0].astype(jnp.int32)
    dst = edge_index[1].astype(jnp.int32)
    et = edge_types.astype(jnp.int32)

    owner = dst // HALF
    lrow = et * HALF_PAD + (dst % HALF)
    srcs = jnp.stack([jnp.where(owner == c, src, ZR) for c in range(NSC)])
    lidx = jnp.stack([jnp.where(owner == c, lrow, 0) for c in range(NSC)])
    ow = jnp.stack([(owner == c).astype(jnp.int32) for c in range(NSC)])

    def chunked(a, pad_value):
        return jnp.pad(a.reshape(NSC, NSUB, EC),
                       ((0, 0), (0, 0), (0, NCHP * CH - EC)),
                       constant_values=pad_value
                       ).reshape(NSC, NSUB, NCHP, CH)

    srcw = chunked(srcs, ZR)
    lidxw = chunked(lidx, 0)
    oww = chunked(ow, 0)

    zeros_big = jnp.zeros((ZROWS, D_H), jnp.float32)
    ones2 = jnp.concatenate([jnp.zeros((1, D_H), jnp.float32),
                             jnp.ones((1, D_H), jnp.float32),
                             jnp.zeros((6, D_H), jnp.float32)])

    cnt = _sc_aggregate(ones2, oww, lidxw, zeros_big)

    p = params
    xp = jnp.pad(x, ((0, NXP - N), (0, 0)))
    h = _tc_proj(xp, p['Wp'], p['bp'].reshape(1, D_H))
    for i in range(NL):
        agg = _sc_aggregate(h, srcw, lidxw, zeros_big)
        bnp = p['bn'][i] if i < NL - 1 else None
        h = _tc_layer(h, agg, cnt, p['layers'][i], bnp)
    return _tc_head(h, p['Wr1'], p['br1'], p['Wr2'], p['br2'])


# speedup vs baseline: 13.6864x; 13.0324x over previous
import functools
import math

import jax
import jax.numpy as jnp
from jax import lax
from jax.experimental import pallas as pl
from jax.experimental.pallas import tpu as pltpu
from jax.experimental.pallas import tpu_sc as plsc

N = 10000
E = 160000
D_IN = 256
D_H = 128
NL = 3
NT = 2

NSC = 2
NSUB = 16
HALF = N // NSC
HALF_PAD = 5120
CPR = HALF_PAD // NSUB
ROWS_PAD = NT * HALF_PAD
ZROWS = ROWS_PAD // NSUB
EC = E // NSUB
CH = 128
NBUF = 2
SIB = 4
LIB = 2
NCHUNK = 80
NCHP = NCHUNK + SIB
ECP = NCHUNK * CH

BLK = 1000
NXP = 11000
NBP = NXP // BLK
ZR = N
OT = 1024
BN_SCALE = 1.0 / math.sqrt(1.0 + 1e-5)

_mesh = plsc.VectorSubcoreMesh(core_axis_name="c", subcore_axis_name="s")


@functools.partial(
    pl.kernel,
    mesh=_mesh,
    out_type=jax.ShapeDtypeStruct((NSC, NT, HALF_PAD, D_H), jnp.float32),
    scratch_types=[
        pltpu.VMEM((NCHP, CH), jnp.int32),
        pltpu.VMEM((NCHP, CH), jnp.int32),
        pltpu.VMEM((CH, D_H), jnp.float32),
        pltpu.VMEM_SHARED((ROWS_PAD, D_H), jnp.float32),
        pltpu.SemaphoreType.DMA,
    ],
)
def _sc_aggregate(h_hbm, src_hbm, lidx_hbm, zeros_hbm, out_hbm,
                  src_v, lidx_v, rows_v, ash, sem):
    c = lax.axis_index("c")
    s = lax.axis_index("s")
    pltpu.sync_copy(zeros_hbm, ash.at[pl.ds(s * ZROWS, ZROWS)])
    pltpu.sync_copy(src_hbm.at[c, s], src_v)
    pltpu.sync_copy(lidx_hbm.at[c, s], lidx_v)
    plsc.subcore_barrier()

    def body(j, carry):
        pltpu.async_copy(h_hbm.at[src_v.at[j]], rows_v, sem).wait()
        pltpu.sync_copy(rows_v, ash.at[lidx_v.at[j]], add=True)
        return carry

    lax.fori_loop(0, NCHUNK, body, 0)
    plsc.subcore_barrier()
    for t in range(NT):
        pltpu.sync_copy(ash.at[pl.ds(t * HALF_PAD + s * CPR, CPR)],
                        out_hbm.at[c, t, pl.ds(s * CPR, CPR)])


def _dot(a, b):
    return lax.dot_general(a, b, (((1,), (1,)), ((), ())),
                           preferred_element_type=jnp.float32)


def _tc_proj_body(x_ref, wp_ref, bp_ref, out_ref):
    i = pl.program_id(0)

    @pl.when(i < NBP - 1)
    def _():
        out_ref[...] = _dot(x_ref[...], wp_ref[...]) + bp_ref[...]

    @pl.when(i == NBP - 1)
    def _():
        out_ref[...] = jnp.zeros((BLK, D_H), jnp.float32)


def _tc_proj(x, wp, bp):
    return pl.pallas_call(
        _tc_proj_body,
        grid=(NBP,),
        in_specs=[
            pl.BlockSpec((BLK, D_IN), lambda i: (i, 0)),
            pl.BlockSpec((D_H, D_IN), lambda i: (0, 0)),
            pl.BlockSpec((1, D_H), lambda i: (0, 0)),
        ],
        out_specs=pl.BlockSpec((BLK, D_H), lambda i: (i, 0)),
        out_shape=jax.ShapeDtypeStruct((NXP, D_H), jnp.float32),
    )(x, wp, bp)


def _tc_layer_body(h_ref, a0_ref, a1_ref, c0_ref, c1_ref,
                   we0_ref, we1_ref, ws_ref,
                   be0_ref, be1_ref, bs_ref, gm_ref, bt_ref,
                   out_ref, *, bn):
    i = pl.program_id(0)

    @pl.when(i < NBP - 1)
    def _():
        acc = (_dot(a0_ref[0, 0], we0_ref[...]) +
               _dot(a1_ref[0, 0], we1_ref[...]) +
               _dot(h_ref[...], ws_ref[...]))
        c0 = c0_ref[0, 0][:, 0:1]
        c1 = c1_ref[0, 0][:, 0:1]
        acc = acc + c0 * be0_ref[...] + c1 * be1_ref[...] + bs_ref[...]
        deg = c0 + c1
        deg = jnp.where(deg == 0.0, 1.0, deg)
        h = jnp.maximum(acc / deg, 0.0)
        if bn:
            h = h * (gm_ref[...] * BN_SCALE) + bt_ref[...]
            h = jnp.maximum(h, 0.0)
        out_ref[...] = h

    @pl.when(i == NBP - 1)
    def _():
        out_ref[...] = jnp.zeros((BLK, D_H), jnp.float32)


_NBH = HALF // BLK


def _tc_layer(h, agg, cnt, lp, bnp):
    bn = bnp is not None
    gm = bnp['gamma'] if bn else lp['bs']
    bt = bnp['beta'] if bn else lp['bs']
    row = lambda v: v.reshape(1, D_H)
    full = lambda: pl.BlockSpec((D_H, D_H), lambda i: (0, 0))
    vec = lambda: pl.BlockSpec((1, D_H), lambda i: (0, 0))
    piece = lambda t: pl.BlockSpec(
        (1, 1, BLK, D_H),
        lambda i, t=t: (jnp.minimum(i // _NBH, NSC - 1), t,
                        jnp.minimum(i % _NBH, _NBH - 1), 0))
    return pl.pallas_call(
        functools.partial(_tc_layer_body, bn=bn),
        grid=(NBP,),
        in_specs=[pl.BlockSpec((BLK, D_H), lambda i: (i, 0)),
                  piece(0), piece(1), piece(0), piece(1),
                  full(), full(), full(),
                  vec(), vec(), vec(), vec(), vec()],
        out_specs=pl.BlockSpec((BLK, D_H), lambda i: (i, 0)),
        out_shape=jax.ShapeDtypeStruct((NXP, D_H), jnp.float32),
    )(h, agg, agg, cnt, cnt,
      lp['We'][0], lp['We'][1], lp['Ws'],
      row(lp['be'][0]), row(lp['be'][1]), row(lp['bs']), row(gm), row(bt))


def _tc_head_body(h_ref, wr1_ref, br1_ref, wr2_ref, br2_ref, out_ref, acc_ref):
    i = pl.program_id(0)

    @pl.when(i == 0)
    def _():
        acc_ref[...] = jnp.zeros((8, D_H), jnp.float32)

    acc_ref[...] = acc_ref[...] + jnp.sum(h_ref[...], axis=0, keepdims=True)

    @pl.when(i == NBP - 1)
    def _():
        g = acc_ref[0:1, :] * (1.0 / N)
        z = jnp.maximum(_dot(g, wr1_ref[...]) + br1_ref[...], 0.0)
        out_ref[...] = _dot(z, wr2_ref[...]) + br2_ref[...]


def _tc_head(h, wr1, br1, wr2, br2):
    return pl.pallas_call(
        _tc_head_body,
        grid=(NBP,),
        in_specs=[
            pl.BlockSpec((BLK, D_H), lambda i: (i, 0)),
            pl.BlockSpec((D_H, D_H), lambda i: (0, 0)),
            pl.BlockSpec((1, D_H), lambda i: (0, 0)),
            pl.BlockSpec((D_H, D_H), lambda i: (0, 0)),
            pl.BlockSpec((1, D_H), lambda i: (0, 0)),
        ],
        out_specs=pl.BlockSpec((1, D_H), lambda i: (0, 0)),
        out_shape=jax.ShapeDtypeStruct((1, D_H), jnp.float32),
        scratch_shapes=[pltpu.VMEM((8, D_H), jnp.float32)],
    )(h, wr1, br1.reshape(1, D_H), wr2, br2.reshape(1, D_H))


def kernel(x, edge_index, edge_types, params):
    src = edge_index[0].astype(jnp.int32)
    dst = edge_index[1].astype(jnp.int32)
    et = edge_types.astype(jnp.int32)

    owner = dst // HALF
    lrow = et * HALF_PAD + (dst % HALF)
    spread = jnp.arange(E, dtype=jnp.int32)
    zrow = ZR + spread % (NXP - N)
    srcs = jnp.stack([jnp.where(owner == c, src, zrow) for c in range(NSC)])
    lidx = jnp.stack([jnp.where(owner == c, lrow, 0) for c in range(NSC)])
    ow = jnp.stack([jnp.where(owner == c, OT + spread % OT, spread % OT)
                    for c in range(NSC)])

    def chunked(a, pad_value):
        return jnp.pad(a.reshape(NSC, NSUB, EC),
                       ((0, 0), (0, 0), (0, NCHP * CH - EC)),
                       constant_values=pad_value
                       ).reshape(NSC, NSUB, NCHP, CH)

    srcw = chunked(srcs, ZR)
    lidxw = chunked(lidx, 0)
    oww = chunked(ow, 0)

    zeros_big = jnp.zeros((ZROWS, D_H), jnp.float32)
    ones2 = jnp.concatenate([jnp.zeros((OT, D_H), jnp.float32),
                             jnp.ones((OT, D_H), jnp.float32)])

    cnt = _sc_aggregate(ones2, oww, lidxw, zeros_big)

    p = params
    xp = jnp.pad(x, ((0, NXP - N), (0, 0)))
    h = _tc_proj(xp, p['Wp'], p['bp'].reshape(1, D_H))
    for i in range(NL):
        agg = _sc_aggregate(h, srcw, lidxw, zeros_big)
        bnp = p['bn'][i] if i < NL - 1 else None
        h = _tc_layer(h, agg, cnt, p['layers'][i], bnp)
    return _tc_head(h, p['Wr1'], p['br1'], p['Wr2'], p['br2'])


# speedup vs baseline: 32.4655x; 2.3721x over previous
import functools
import math

import jax
import jax.numpy as jnp
from jax import lax
from jax.experimental import pallas as pl
from jax.experimental.pallas import tpu as pltpu
from jax.experimental.pallas import tpu_sc as plsc

N = 10000
E = 160000
D_IN = 256
D_H = 128
NL = 3
NT = 2

NSC = 2
NSUB = 16
HALF = N // NSC
HALF_PAD = 5120
CPR = HALF_PAD // NSUB
ROWS_PAD = NT * HALF_PAD
ZROWS = ROWS_PAD // NSUB
EC = E // NSUB
CH = 128
NBUF = 2
SIB = 4
LIB = 2
NCHUNK = 80
NCHP = NCHUNK + SIB
ECP = NCHUNK * CH

BLK = 1000
NXP = 11000
NBP = NXP // BLK
ZR = N
OT = 1024
BN_SCALE = 1.0 / math.sqrt(1.0 + 1e-5)

_mesh = plsc.VectorSubcoreMesh(core_axis_name="c", subcore_axis_name="s")


@functools.partial(
    pl.kernel,
    mesh=_mesh,
    out_type=jax.ShapeDtypeStruct((NSC, NT, HALF_PAD, D_H), jnp.float32),
    scratch_types=[
        pltpu.VMEM((NCHP, CH), jnp.int32),
        pltpu.VMEM((NCHP, CH), jnp.int32),
        pltpu.VMEM((CH, D_H), jnp.float32),
        pltpu.VMEM_SHARED((ROWS_PAD, D_H), jnp.float32),
        pltpu.SemaphoreType.DMA,
    ],
)
def _sc_aggregate(h_hbm, src_hbm, lidx_hbm, zeros_hbm, out_hbm,
                  src_v, lidx_v, rows_v, ash, sem):
    c = lax.axis_index("c")
    s = lax.axis_index("s")
    pltpu.sync_copy(zeros_hbm, ash.at[pl.ds(s * ZROWS, ZROWS)])
    pltpu.sync_copy(src_hbm.at[c, s], src_v)
    pltpu.sync_copy(lidx_hbm.at[c, s], lidx_v)
    plsc.subcore_barrier()

    def body(j, carry):
        pltpu.async_copy(h_hbm.at[src_v.at[j]], rows_v, sem).wait()
        pltpu.sync_copy(rows_v, ash.at[lidx_v.at[j]], add=True)
        return carry

    lax.fori_loop(0, NCHUNK, body, 0)
    plsc.subcore_barrier()
    for t in range(NT):
        pltpu.sync_copy(ash.at[pl.ds(t * HALF_PAD + s * CPR, CPR)],
                        out_hbm.at[c, t, pl.ds(s * CPR, CPR)])


def _dot(a, b):
    return lax.dot_general(a, b, (((1,), (1,)), ((), ())),
                           preferred_element_type=jnp.float32)


def _tc_proj_body(x_ref, wp_ref, bp_ref, out_ref):
    i = pl.program_id(0)

    @pl.when(i < NBP - 1)
    def _():
        out_ref[...] = _dot(x_ref[...], wp_ref[...]) + bp_ref[...]

    @pl.when(i == NBP - 1)
    def _():
        out_ref[...] = jnp.zeros((BLK, D_H), jnp.float32)


def _tc_proj(x, wp, bp):
    return pl.pallas_call(
        _tc_proj_body,
        grid=(NBP,),
        in_specs=[
            pl.BlockSpec((BLK, D_IN), lambda i: (i, 0)),
            pl.BlockSpec((D_H, D_IN), lambda i: (0, 0)),
            pl.BlockSpec((1, D_H), lambda i: (0, 0)),
        ],
        out_specs=pl.BlockSpec((BLK, D_H), lambda i: (i, 0)),
        out_shape=jax.ShapeDtypeStruct((NXP, D_H), jnp.float32),
    )(x, wp, bp)


def _tc_layer_body(h_ref, a0_ref, a1_ref, c0_ref, c1_ref,
                   we0_ref, we1_ref, ws_ref,
                   be0_ref, be1_ref, bs_ref, gm_ref, bt_ref,
                   out_ref, *, bn):
    i = pl.program_id(0)

    @pl.when(i < NBP - 1)
    def _():
        acc = (_dot(a0_ref[0, 0], we0_ref[...]) +
               _dot(a1_ref[0, 0], we1_ref[...]) +
               _dot(h_ref[...], ws_ref[...]))
        c0 = c0_ref[0, 0][:, 0:1]
        c1 = c1_ref[0, 0][:, 0:1]
        acc = acc + c0 * be0_ref[...] + c1 * be1_ref[...] + bs_ref[...]
        deg = c0 + c1
        deg = jnp.where(deg == 0.0, 1.0, deg)
        h = jnp.maximum(acc / deg, 0.0)
        if bn:
            h = h * (gm_ref[...] * BN_SCALE) + bt_ref[...]
            h = jnp.maximum(h, 0.0)
        out_ref[...] = h

    @pl.when(i == NBP - 1)
    def _():
        out_ref[...] = jnp.zeros((BLK, D_H), jnp.float32)


_NBH = HALF // BLK


def _tc_layer(h, agg, cnt, lp, bnp):
    bn = bnp is not None
    gm = bnp['gamma'] if bn else lp['bs']
    bt = bnp['beta'] if bn else lp['bs']
    row = lambda v: v.reshape(1, D_H)
    full = lambda: pl.BlockSpec((D_H, D_H), lambda i: (0, 0))
    vec = lambda: pl.BlockSpec((1, D_H), lambda i: (0, 0))
    piece = lambda t: pl.BlockSpec(
        (1, 1, BLK, D_H),
        lambda i, t=t: (jnp.minimum(i // _NBH, NSC - 1), t,
                        jnp.minimum(i % _NBH, _NBH - 1), 0))
    return pl.pallas_call(
        functools.partial(_tc_layer_body, bn=bn),
        grid=(NBP,),
        in_specs=[pl.BlockSpec((BLK, D_H), lambda i: (i, 0)),
                  piece(0), piece(1), piece(0), piece(1),
                  full(), full(), full(),
                  vec(), vec(), vec(), vec(), vec()],
        out_specs=pl.BlockSpec((BLK, D_H), lambda i: (i, 0)),
        out_shape=jax.ShapeDtypeStruct((NXP, D_H), jnp.float32),
    )(h, agg, agg, cnt, cnt,
      lp['We'][0], lp['We'][1], lp['Ws'],
      row(lp['be'][0]), row(lp['be'][1]), row(lp['bs']), row(gm), row(bt))


def _tc_head_body(h_ref, wr1_ref, br1_ref, wr2_ref, br2_ref, out_ref, acc_ref):
    i = pl.program_id(0)

    @pl.when(i == 0)
    def _():
        acc_ref[...] = jnp.zeros((8, D_H), jnp.float32)

    acc_ref[...] = acc_ref[...] + jnp.sum(h_ref[...], axis=0, keepdims=True)

    @pl.when(i == NBP - 1)
    def _():
        g = acc_ref[0:1, :] * (1.0 / N)
        z = jnp.maximum(_dot(g, wr1_ref[...]) + br1_ref[...], 0.0)
        out_ref[...] = _dot(z, wr2_ref[...]) + br2_ref[...]


def _tc_head(h, wr1, br1, wr2, br2):
    return pl.pallas_call(
        _tc_head_body,
        grid=(NBP,),
        in_specs=[
            pl.BlockSpec((BLK, D_H), lambda i: (i, 0)),
            pl.BlockSpec((D_H, D_H), lambda i: (0, 0)),
            pl.BlockSpec((1, D_H), lambda i: (0, 0)),
            pl.BlockSpec((D_H, D_H), lambda i: (0, 0)),
            pl.BlockSpec((1, D_H), lambda i: (0, 0)),
        ],
        out_specs=pl.BlockSpec((1, D_H), lambda i: (0, 0)),
        out_shape=jax.ShapeDtypeStruct((1, D_H), jnp.float32),
        scratch_shapes=[pltpu.VMEM((8, D_H), jnp.float32)],
    )(h, wr1, br1.reshape(1, D_H), wr2, br2.reshape(1, D_H))


def kernel(x, edge_index, edge_types, params):
    src = edge_index[0].astype(jnp.int32)
    dst = edge_index[1].astype(jnp.int32)
    et = edge_types.astype(jnp.int32)

    owner = dst // HALF
    lrow = et * HALF_PAD + (dst % HALF)
    spread = jnp.arange(E, dtype=jnp.int32)
    zrow = ZR + spread % (NXP - N)
    srcs = jnp.stack([jnp.where(owner == c, src, zrow) for c in range(NSC)])
    lidx = jnp.stack([jnp.where(owner == c, lrow, spread % ROWS_PAD)
                      for c in range(NSC)])
    ow = jnp.stack([jnp.where(owner == c, OT + spread % OT, spread % OT)
                    for c in range(NSC)])

    padn = NCHP * CH - EC
    pads = jnp.arange(padn, dtype=jnp.int32)

    def chunked(a, padv):
        tail = jnp.broadcast_to(padv, (NSC, NSUB, padn))
        return jnp.concatenate([a.reshape(NSC, NSUB, EC), tail],
                               axis=-1).reshape(NSC, NSUB, NCHP, CH)

    srcw = chunked(srcs, ZR + pads % (NXP - N))
    lidxw = chunked(lidx, pads % ROWS_PAD)
    oww = chunked(ow, pads % OT)

    zeros_big = jnp.zeros((ZROWS, D_H), jnp.float32)
    ones2 = jnp.concatenate([jnp.zeros((OT, D_H), jnp.float32),
                             jnp.ones((OT, D_H), jnp.float32)])

    cnt = _sc_aggregate(ones2, oww, lidxw, zeros_big)

    p = params
    xp = jnp.pad(x, ((0, NXP - N), (0, 0)))
    h = _tc_proj(xp, p['Wp'], p['bp'].reshape(1, D_H))
    for i in range(NL):
        agg = _sc_aggregate(h, srcw, lidxw, zeros_big)
        bnp = p['bn'][i] if i < NL - 1 else None
        h = _tc_layer(h, agg, cnt, p['layers'][i], bnp)
    return _tc_head(h, p['Wr1'], p['br1'], p['Wr2'], p['br2'])
